# baseline (device time: 48281 ns/iter reference)
import jax
import jax.numpy as jnp
from jax import lax
from jax.experimental import pallas as pl
from jax.experimental.pallas import tpu as pltpu

B = 512
NSLOT = 2
BF = jnp.bfloat16


def kernel(x):
    m, n = x.shape
    assert m % B == 0 and B % 8 == 0
    nc = m // B

    def body(x_hbm, out_hbm, xbuf, obuf, rtile, ctile, petile, prtile,
             rsend, csend, rhalo, chalo, oedge, orow,
             load_sems, store_sems, edge_sems, patch_sems,
             send_sems, recv_sems):
        my_x = lax.axis_index("x")
        my_y = lax.axis_index("y")

        bsem = pltpu.get_barrier_semaphore()
        pl.semaphore_signal(bsem, inc=1, device_id=(1 - my_x, my_y),
                            device_id_type=pl.DeviceIdType.MESH)
        pl.semaphore_signal(bsem, inc=1, device_id=(my_x, 1 - my_y),
                            device_id_type=pl.DeviceIdType.MESH)
        pl.semaphore_wait(bsem, 2)

        rt_off = (1 - my_x) * (m - 8)
        ct_off = (1 - my_y) * (n - 128)
        pe_off = (1 - my_y) * (n - 256)
        pr_off = (1 - my_x) * (m - 16)
        cp_rt = pltpu.make_async_copy(
            x_hbm.at[pl.ds(rt_off, 8), :], rtile, edge_sems.at[0])
        cp_ct = pltpu.make_async_copy(
            x_hbm.at[:, pl.ds(ct_off, 128)], ctile, edge_sems.at[1])
        cp_pe = pltpu.make_async_copy(
            x_hbm.at[:, pl.ds(pe_off, 256)], petile, patch_sems.at[0])
        cp_pr = pltpu.make_async_copy(
            x_hbm.at[pl.ds(pr_off, 16), :], prtile, patch_sems.at[1])
        cp_rt.start()
        cp_ct.start()
        cp_pe.start()
        cp_pr.start()

        cp_rt.wait()
        rt = rtile[...]
        rsend[...] = jnp.where(my_x == 0, rt[7:8, :], rt[0:1, :])
        cp_ct.wait()
        ct = ctile[...]
        csend[...] = jnp.where(my_y == 0, ct[:, 127:128], ct[:, 0:1])

        rdma_row = pltpu.make_async_remote_copy(
            src_ref=rsend, dst_ref=rhalo,
            send_sem=send_sems.at[0], recv_sem=recv_sems.at[0],
            device_id=(1 - my_x, my_y), device_id_type=pl.DeviceIdType.MESH)
        rdma_col = pltpu.make_async_remote_copy(
            src_ref=csend, dst_ref=chalo,
            send_sem=send_sems.at[1], recv_sem=recv_sems.at[1],
            device_id=(my_x, 1 - my_y), device_id_type=pl.DeviceIdType.MESH)
        rdma_row.start()
        rdma_col.start()

        def make_load(c):
            lo = max(c * B - 8, 0)
            hi = min(c * B + B + 8, m)
            off = 8 if c == 0 else 0
            return pltpu.make_async_copy(
                x_hbm.at[pl.ds(lo, hi - lo), :],
                xbuf.at[c % NSLOT, pl.ds(off, hi - lo), :],
                load_sems.at[c % NSLOT])

        loads = {}
        for c in range(NSLOT):
            loads[c] = make_load(c)
            loads[c].start()

        stores = {}
        for c in range(nc):
            slot = c % NSLOT
            loads[c].wait()
            if c >= NSLOT:
                stores[c - NSLOT].wait()

            xb = xbuf[slot].astype(BF)
            center = xb[8:B + 8, :]
            up = xb[7:B + 7, :]
            down = xb[9:B + 9, :]
            ch = center[:, 0:1]
            left = jnp.concatenate([ch, center[:, :n - 1]], axis=1)
            right = jnp.concatenate([center[:, 1:], ch], axis=1)
            res = BF(0.5) * center + BF(0.125) * ((up + down) + (left + right))

            obuf[slot, :, :] = res

            @pl.when(my_y == 0)
            def _():
                obuf[slot, :, 0:1] = center[:, 0:1]

            @pl.when(my_y == 1)
            def _():
                obuf[slot, :, n - 1:n] = center[:, n - 1:n]

            if c == 0:
                @pl.when(my_x == 0)
                def _():
                    obuf[slot, 0:1, :] = center[0:1, :]
            if c == nc - 1:
                @pl.when(my_x == 1)
                def _():
                    obuf[slot, B - 1:B, :] = center[B - 1:B, :]

            stores[c] = pltpu.make_async_copy(
                obuf.at[slot],
                out_hbm.at[pl.ds(c * B, B), :],
                store_sems.at[slot])
            stores[c].start()
            if c + NSLOT < nc:
                loads[c + NSLOT] = make_load(c + NSLOT)
                loads[c + NSLOT].start()

        for c in range(max(nc - NSLOT, 0), nc):
            stores[c].wait()

        rdma_row.wait()
        rdma_col.wait()
        cp_pe.wait()
        cp_pr.wait()

        rh = rhalo[...].astype(BF)
        chv = chalo[...].astype(BF)
        W = petile[...].astype(BF)
        ce = jnp.where(my_y == 0, W[:, 128:256], W[:, 0:128])
        rh_e = jnp.where(my_y == 0, rh[:, n - 128:n], rh[:, 0:128])
        up_e = jnp.concatenate([rh_e, ce[:m - 1, :]], axis=0)
        down_e = jnp.concatenate([ce[1:, :], rh_e], axis=0)
        left_e = jnp.where(
            my_y == 0, W[:, 127:255],
            jnp.concatenate([chv, W[:, 0:127]], axis=1))
        right_e = jnp.where(
            my_y == 0, jnp.concatenate([W[:, 129:256], chv], axis=1),
            W[:, 1:129])
        pe = BF(0.5) * ce + BF(0.125) * ((up_e + down_e) + (left_e + right_e))
        oedge[...] = pe
        @pl.when(my_x == 0)
        def _():
            oedge[0:1, :] = ce[0:1, :]

        @pl.when(my_x == 1)
        def _():
            oedge[m - 1:m, :] = ce[m - 1:m, :]

        st_e = pltpu.make_async_copy(
            oedge, out_hbm.at[:, pl.ds(ct_off, 128)], patch_sems.at[0])
        st_e.start()

        pr = prtile[...].astype(BF)
        center_r = jnp.where(my_x == 1, pr[0:8, :], pr[8:16, :])
        up_r = jnp.where(
            my_x == 1,
            jnp.concatenate([rh, pr[0:7, :]], axis=0),
            pr[7:15, :])
        down_r = jnp.where(
            my_x == 1, pr[1:9, :],
            jnp.concatenate([pr[9:16, :], rh], axis=0))
        chr_ = jnp.where(my_x == 1, chv[0:8, :], chv[m - 8:m, :])
        left_r = jnp.concatenate([chr_, center_r[:, :n - 1]], axis=1)
        right_r = jnp.concatenate([center_r[:, 1:], chr_], axis=1)
        prow = BF(0.5) * center_r + BF(0.125) * (
            (up_r + down_r) + (left_r + right_r))
        orow[...] = prow
        @pl.when(my_y == 0)
        def _():
            orow[:, 0:1] = center_r[:, 0:1]

        @pl.when(my_y == 1)
        def _():
            orow[:, n - 1:n] = center_r[:, n - 1:n]

        st_r = pltpu.make_async_copy(
            orow, out_hbm.at[pl.ds(rt_off, 8), :], patch_sems.at[1])
        st_r.start()
        st_e.wait()
        st_r.wait()

    return pl.pallas_call(
        body,
        out_shape=jax.ShapeDtypeStruct((m, n), jnp.bfloat16),
        in_specs=[pl.BlockSpec(memory_space=pl.ANY)],
        out_specs=pl.BlockSpec(memory_space=pl.ANY),
        scratch_shapes=[
            pltpu.VMEM((NSLOT, B + 16, n), jnp.float32),
            pltpu.VMEM((NSLOT, B, n), jnp.bfloat16),
            pltpu.VMEM((8, n), jnp.float32),
            pltpu.VMEM((m, 128), jnp.float32),
            pltpu.VMEM((m, 256), jnp.float32),
            pltpu.VMEM((16, n), jnp.float32),
            pltpu.VMEM((1, n), jnp.float32),
            pltpu.VMEM((m, 1), jnp.float32),
            pltpu.VMEM((1, n), jnp.float32),
            pltpu.VMEM((m, 1), jnp.float32),
            pltpu.VMEM((m, 128), jnp.bfloat16),
            pltpu.VMEM((8, n), jnp.bfloat16),
            pltpu.SemaphoreType.DMA((NSLOT,)),
            pltpu.SemaphoreType.DMA((NSLOT,)),
            pltpu.SemaphoreType.DMA((2,)),
            pltpu.SemaphoreType.DMA((2,)),
            pltpu.SemaphoreType.DMA((2,)),
            pltpu.SemaphoreType.DMA((2,)),
        ],
        compiler_params=pltpu.CompilerParams(
            collective_id=0, vmem_limit_bytes=64 * 1024 * 1024),
    )(x)
